# baseline (speedup 1.0000x reference)
import functools

import jax
import jax.numpy as jnp
from jax.experimental import pallas as pl
from jax.experimental.pallas import tpu as pltpu


def _attn_body(scores_ref, mask_ref, valid_ref, reps_hbm, out_ref,
               buf, sem, *, bt, nb, k_slots, rows):
    k = pl.program_id(0)

    def start_copy(j, slot):
        pltpu.make_async_copy(
            reps_hbm.at[pl.ds(j * bt, bt), :, :],
            buf.at[slot],
            sem.at[slot],
        ).start()

    @pl.when(k == 0)
    def _():
        for j in range(min(k_slots - 1, nb)):
            start_copy(j, j)

    j = k + k_slots - 1

    @pl.when(j < nb)
    def _():
        start_copy(j, j % k_slots)

    pltpu.make_async_copy(
        reps_hbm.at[pl.ds(k * bt, bt), :, :],
        buf.at[k % k_slots],
        sem.at[k % k_slots],
    ).wait()

    r0 = k * bt
    s = scores_ref[pl.ds(r0, bt), :].astype(jnp.float32)
    keep = jnp.logical_and(mask_ref[pl.ds(r0, bt), :], valid_ref[pl.ds(r0, bt), :])
    s = jnp.where(keep, s, jnp.float32(-10000.0))
    mx = jnp.max(s, axis=-1, keepdims=True)
    e = jnp.exp(s - mx)
    att = e / jnp.sum(e, axis=-1, keepdims=True)

    reps_blk = buf.at[k % k_slots]
    for c0 in range(0, bt, rows):
        c1 = c0 + rows
        r = reps_blk[c0:c1, :, :].astype(jnp.float32)
        w = att[c0:c1, :]
        out = jnp.sum(w[:, :, None] * r, axis=1)
        out_ref[pl.ds(r0 + c0, rows), :] = out.astype(out_ref.dtype)


def _pick_bt(B, N, D, itemsize, target_bytes=6 << 20):
    row_bytes = max(1, N * D * itemsize)
    best = 8
    for bt in range(8, B + 1, 8):
        if B % bt or bt * row_bytes > target_bytes:
            continue
        best = bt
    return best


def kernel(sentence_reps, sentence_mask, att_scores, valid_scores):
    B, N, D = sentence_reps.shape
    out_dtype = sentence_reps.dtype
    itemsize = sentence_reps.dtype.itemsize

    bt = _pick_bt(B, N, D, itemsize, target_bytes=6 << 20)
    nb = B // bt
    k_slots = min(4, nb)
    rows = 8 if bt % 8 == 0 else bt

    reps_blk = bt * N * D * itemsize
    needed = k_slots * reps_blk + (12 << 20)

    entry = pl.pallas_call(
        functools.partial(_attn_body, bt=bt, nb=nb, k_slots=k_slots, rows=rows),
        out_shape=jax.ShapeDtypeStruct((B, D), out_dtype),
        grid=(nb,),
        in_specs=[
            pl.BlockSpec((B, N), lambda b: (0, 0)),
            pl.BlockSpec((B, N), lambda b: (0, 0)),
            pl.BlockSpec((B, N), lambda b: (0, 0)),
            pl.BlockSpec(memory_space=pl.ANY),
        ],
        out_specs=pl.BlockSpec((B, D), lambda b: (0, 0)),
        scratch_shapes=[
            pltpu.VMEM((k_slots, bt, N, D), sentence_reps.dtype),
            pltpu.SemaphoreType.DMA((k_slots,)),
        ],
        compiler_params=pltpu.CompilerParams(
            dimension_semantics=("arbitrary",),
            vmem_limit_bytes=int(min(max(needed, 32 << 20), 58 << 20)),
        ),
    )
    return entry(att_scores, sentence_mask, valid_scores, sentence_reps)


# speedup vs baseline: 1.0090x; 1.0090x over previous
import functools

import jax
import jax.numpy as jnp
from jax.experimental import pallas as pl
from jax.experimental.pallas import tpu as pltpu


def _attn_body(scores_ref, mask_ref, valid_ref, reps_ref, out_ref, *, rows):
    bt, n = scores_ref.shape

    s = scores_ref[...].astype(jnp.float32)
    keep = jnp.logical_and(mask_ref[...], valid_ref[...])
    s = jnp.where(keep, s, jnp.float32(-10000.0))
    mx = jnp.max(s, axis=-1, keepdims=True)
    e = jnp.exp(s - mx)
    att = e / jnp.sum(e, axis=-1, keepdims=True)

    for c0 in range(0, bt, rows):
        c1 = c0 + rows
        r = reps_ref[c0:c1, :, :].astype(jnp.float32)
        w = att[c0:c1, :]
        out = jnp.sum(w[:, :, None] * r, axis=1)
        out_ref[c0:c1, :] = out.astype(out_ref.dtype)


def kernel(sentence_reps, sentence_mask, att_scores, valid_scores):
    B, N, D = sentence_reps.shape
    out_dtype = sentence_reps.dtype
    itemsize = sentence_reps.dtype.itemsize

    bt = 48
    if B % 8 == 0 and B < bt:
        bt = B
    grid = (pl.cdiv(B, bt),)

    rows = 8 if bt % 8 == 0 else bt

    reps_blk = bt * N * D * itemsize
    needed = 2 * reps_blk + (8 << 20)

    entry = pl.pallas_call(
        functools.partial(_attn_body, rows=rows),
        out_shape=jax.ShapeDtypeStruct((B, D), out_dtype),
        grid=grid,
        in_specs=[
            pl.BlockSpec((bt, N), lambda b: (b, 0)),
            pl.BlockSpec((bt, N), lambda b: (b, 0)),
            pl.BlockSpec((bt, N), lambda b: (b, 0)),
            pl.BlockSpec((bt, N, D), lambda b: (b, 0, 0)),
        ],
        out_specs=pl.BlockSpec((bt, D), lambda b: (b, 0)),
        compiler_params=pltpu.CompilerParams(
            dimension_semantics=("arbitrary",),
            vmem_limit_bytes=int(min(max(needed, 32 << 20), 58 << 20)),
        ),
    )
    return entry(att_scores, sentence_mask, valid_scores, sentence_reps)


# speedup vs baseline: 1.0225x; 1.0134x over previous
import functools

import jax
import jax.numpy as jnp
from jax.experimental import pallas as pl
from jax.experimental.pallas import tpu as pltpu


def _attn_body(scores_ref, mask_ref, valid_ref, reps_ref, out_ref, *, rows):
    bt, n = scores_ref.shape

    s = scores_ref[...].astype(jnp.float32)
    keep = jnp.logical_and(mask_ref[...], valid_ref[...])
    s = jnp.where(keep, s, jnp.float32(-10000.0))
    mx = jnp.max(s, axis=-1, keepdims=True)
    e = jnp.exp(s - mx)
    att = e / jnp.sum(e, axis=-1, keepdims=True)

    for c0 in range(0, bt, rows):
        c1 = c0 + rows
        r = reps_ref[c0:c1, :, :].astype(jnp.float32)
        w = att[c0:c1, :]
        out = jnp.sum(w[:, :, None] * r, axis=1)
        out_ref[c0:c1, :] = out.astype(out_ref.dtype)


def kernel(sentence_reps, sentence_mask, att_scores, valid_scores):
    B, N, D = sentence_reps.shape
    out_dtype = sentence_reps.dtype
    itemsize = sentence_reps.dtype.itemsize

    bt = 40
    if B % 8 == 0 and B < bt:
        bt = B
    grid = (pl.cdiv(B, bt),)

    rows = 8 if bt % 8 == 0 else bt

    reps_blk = bt * N * D * itemsize
    needed = 2 * reps_blk + (8 << 20)

    entry = pl.pallas_call(
        functools.partial(_attn_body, rows=rows),
        out_shape=jax.ShapeDtypeStruct((B, D), out_dtype),
        grid=grid,
        in_specs=[
            pl.BlockSpec((bt, N), lambda b: (b, 0)),
            pl.BlockSpec((bt, N), lambda b: (b, 0)),
            pl.BlockSpec((bt, N), lambda b: (b, 0)),
            pl.BlockSpec((bt, N, D), lambda b: (b, 0, 0)),
        ],
        out_specs=pl.BlockSpec((bt, D), lambda b: (b, 0)),
        compiler_params=pltpu.CompilerParams(
            dimension_semantics=("arbitrary",),
            vmem_limit_bytes=int(min(max(needed, 32 << 20), 58 << 20)),
        ),
    )
    return entry(att_scores, sentence_mask, valid_scores, sentence_reps)
